# Initial kernel scaffold; baseline (speedup 1.0000x reference)
import jax
import jax.numpy as jnp
from jax.experimental import pallas as pl


def kernel(input, graph):
    raise NotImplementedError("write your pallas kernel here")


# speedup vs baseline: 1.8707x; 1.8707x over previous
import functools

import jax
import jax.numpy as jnp
from jax import lax
from jax.experimental import pallas as pl
from jax.experimental.pallas import tpu as pltpu
from jax.experimental.pallas import tpu_sc as plsc

_B, _T, _D = 8, 4096, 512
_P = 4
_S = _T // _P
_NW = 32
_L = 16
_DPW = _D // _NW
_ROWS = _T + 8


def _body(x_hbm, g_hbm, out_hbm, csbuf, outbuf, gbuf):
    nc = 2
    wid = lax.axis_index("s") * nc + lax.axis_index("c")
    d0 = wid * _DPW
    lane = lax.iota(jnp.int32, _L)
    zero_i = jnp.zeros((_L,), jnp.int32)
    one_i = jnp.ones((_L,), jnp.int32)
    zero_f = jnp.zeros((_L,), jnp.float32)

    for b in range(_B):
        pltpu.sync_copy(x_hbm.at[b, :, pl.ds(d0, _DPW)],
                        csbuf.at[pl.ds(8, _T)])
        pltpu.sync_copy(g_hbm.at[b], gbuf)
        csbuf[7] = zero_f

        def scan_body(i, carry):
            base = 8 + i * 8
            c = carry
            mid = carry
            for j in range(8):
                c = c + csbuf[base + j]
                csbuf[base + j] = c
                if j == 3:
                    outbuf[4 * i + 1] = (c - carry) * 0.25
                    mid = c
            outbuf[4 * i + 3] = (c - mid) * 0.25
            return c

        lax.fori_loop(0, _T // 8, scan_body, zero_f, unroll=False)

        def shot_body(g, _):
            svec = g * _L + lane
            avec = plsc.load_gather(gbuf, [svec, zero_i])
            bvec = plsc.load_gather(gbuf, [svec, one_i])
            inv = 1.0 / (bvec - avec + 1).astype(jnp.float32)
            lo = avec + 7
            hi = bvec + 8
            orow = svec * 2
            for k in range(_DPW):
                dd = (lane + k) & (_L - 1)
                sa = plsc.load_gather(csbuf, [lo, dd])
                sb = plsc.load_gather(csbuf, [hi, dd])
                plsc.store_scatter(outbuf, [orow, dd],
                                   (sb - sa) * inv + 0.006)
            return 0

        lax.fori_loop(0, _S // _L, shot_body, 0, unroll=False)

        pltpu.sync_copy(outbuf, out_hbm.at[b, :, pl.ds(d0, _DPW)])


@jax.jit
def _pool(x, graph):
    mesh = plsc.VectorSubcoreMesh(core_axis_name="c", subcore_axis_name="s")
    fn = pl.kernel(
        _body,
        out_type=jax.ShapeDtypeStruct((_B, 2 * _S, _D), jnp.float32),
        mesh=mesh,
        scratch_types=[
            pltpu.VMEM((_ROWS, _DPW), jnp.float32),
            pltpu.VMEM((2 * _S, _DPW), jnp.float32),
            pltpu.VMEM((_S, 2), jnp.int32),
        ],
        compiler_params=pltpu.CompilerParams(use_tc_tiling_on_sc=False,
                                             needs_layout_passes=False),
    )
    return fn(x, graph)


def kernel(input, graph):
    return _pool(input, graph.astype(jnp.int32))


# speedup vs baseline: 2.7422x; 1.4659x over previous
import functools

import jax
import jax.numpy as jnp
from jax import lax
from jax.experimental import pallas as pl
from jax.experimental.pallas import tpu as pltpu
from jax.experimental.pallas import tpu_sc as plsc

_B, _T, _D = 8, 4096, 512
_P = 4
_S = _T // _P
_NW = 32
_L = 16
_DPW = _D // _NW
_ROWS = _T + 8


def _body(x_hbm, g_hbm, out_hbm, csbuf, outbuf, gbuf, offtab):
    nc = 2
    wid = lax.axis_index("s") * nc + lax.axis_index("c")
    d0 = wid * _DPW
    lane = lax.iota(jnp.int32, _L)
    zero_i = jnp.zeros((_L,), jnp.int32)
    one_i = jnp.ones((_L,), jnp.int32)
    zero_f = jnp.zeros((_L,), jnp.float32)

    nch = 8
    chrows = _T // nch
    chsh = 9

    def batch_body(b, _):
        pltpu.sync_copy(x_hbm.at[b, :, pl.ds(d0, _DPW)],
                        csbuf.at[pl.ds(8, _T)])
        pltpu.sync_copy(g_hbm.at[b], gbuf)
        csbuf[7] = zero_f

        @plsc.parallel_loop(0, chrows // 8, carry=(zero_f,) * nch)
        def scan_body(i, carry):
            c = list(carry)
            for k in range(nch):
                base = 8 + k * chrows + i * 8
                ck = c[k]
                start = ck
                mid = ck
                orow = 4 * (k * (chrows // 8) + i)
                for j in range(8):
                    ck = ck + csbuf[base + j]
                    csbuf[base + j] = ck
                    if j == 3:
                        outbuf[orow + 1] = (ck - start) * 0.25
                        mid = ck
                outbuf[orow + 3] = (ck - mid) * 0.25
                c[k] = ck
            return tuple(c)

        carries = scan_body

        offtab[0] = zero_f
        offtab[1] = zero_f
        acc = carries[0]
        for k in range(1, nch):
            offtab[k + 1] = acc
            acc = acc + carries[k]

        @plsc.parallel_loop(0, (_S // _L) * 2)
        def shot_body(i):
            g = i >> 1
            svec = g * _L + lane
            avec = plsc.load_gather(gbuf, [svec, zero_i])
            bvec = plsc.load_gather(gbuf, [svec, one_i])
            inv = 1.0 / (bvec - avec + 1).astype(jnp.float32)
            lo = avec + 7
            hi = bvec + 8
            lo_oi = (avec + (chrows - 1)) >> chsh
            hi_oi = (bvec >> chsh) + 1
            orow = svec * 2
            dd = (lane + ((i & 1) << 3)) & (_L - 1)
            for k in range(_DPW // 2):
                sa = plsc.load_gather(csbuf, [lo, dd])
                sb = plsc.load_gather(csbuf, [hi, dd])
                oa = plsc.load_gather(offtab, [lo_oi, dd])
                ob = plsc.load_gather(offtab, [hi_oi, dd])
                plsc.store_scatter(outbuf, [orow, dd],
                                   ((sb - sa) + (ob - oa)) * inv + 0.006)
                if k < _DPW // 2 - 1:
                    dd = (dd + 1) & (_L - 1)

        pltpu.sync_copy(outbuf, out_hbm.at[b, :, pl.ds(d0, _DPW)])
        return 0

    lax.fori_loop(0, _B, batch_body, 0, unroll=False)


@jax.jit
def _pool(x, graph):
    mesh = plsc.VectorSubcoreMesh(core_axis_name="c", subcore_axis_name="s")
    fn = pl.kernel(
        _body,
        out_type=jax.ShapeDtypeStruct((_B, 2 * _S, _D), jnp.float32),
        mesh=mesh,
        scratch_types=[
            pltpu.VMEM((_ROWS, _DPW), jnp.float32),
            pltpu.VMEM((2 * _S, _DPW), jnp.float32),
            pltpu.VMEM((_S, 2), jnp.int32),
            pltpu.VMEM((9, _L), jnp.float32),
        ],
        compiler_params=pltpu.CompilerParams(use_tc_tiling_on_sc=False,
                                             needs_layout_passes=False,
                                             disable_bounds_checks=True),
    )
    return fn(x, graph)


def kernel(input, graph):
    return _pool(input, graph.astype(jnp.int32))
